# baseline (speedup 1.0000x reference)
import functools

import jax
import jax.numpy as jnp
from jax import lax
from jax.experimental import pallas as pl
from jax.experimental.pallas import tpu as pltpu
from jax.experimental.pallas import tpu_sc as plsc

BATCH = 16384
DIM = 32

_info = plsc.get_sparse_core_info()
NC, NS, NL = _info.num_cores, _info.num_subcores, _info.num_lanes
NW = NC * NS
BPW = BATCH // NW
CHUNK = 128
NCHUNK = BPW // CHUNK


def _body(uid_hbm, iid_hbm, ut_hbm, it_hbm, out_hbm,
          uidx_v, iidx_v, urows_v, irows_v, out_v, gsem):
    wid = lax.axis_index("s") * NC + lax.axis_index("c")
    base = wid * BPW

    pltpu.sync_copy(uid_hbm.at[pl.ds(base, BPW)], uidx_v)
    pltpu.sync_copy(iid_hbm.at[pl.ds(base, BPW)], iidx_v)

    copies = []
    for j in range(NCHUNK):
        sl = pl.ds(j * CHUNK, CHUNK)
        copies.append(pltpu.async_copy(ut_hbm.at[uidx_v.at[sl]], urows_v.at[sl], gsem))
        copies.append(pltpu.async_copy(it_hbm.at[iidx_v.at[sl]], irows_v.at[sl], gsem))
    for c in copies:
        c.wait()

    lane = lax.iota(jnp.int32, NL)
    cols = [jnp.full((NL,), d, jnp.int32) for d in range(DIM)]

    def group(g, carry):
        row = g * NL + lane
        acc = jnp.zeros((NL,), jnp.float32)
        for d in range(DIM):
            u = plsc.load_gather(urows_v, [row, cols[d]])
            v = plsc.load_gather(irows_v, [row, cols[d]])
            acc = acc + u * v
        out_v[pl.ds(g * NL, NL)] = acc
        return carry

    lax.fori_loop(0, BPW // NL, group, 0)

    pltpu.sync_copy(out_v, out_hbm.at[pl.ds(base, BPW)])


def kernel(user_ids, item_ids, user_table, item_table):
    mesh = plsc.VectorSubcoreMesh(core_axis_name="c", subcore_axis_name="s")
    f = pl.kernel(
        _body,
        mesh=mesh,
        out_type=jax.ShapeDtypeStruct((BATCH,), jnp.float32),
        scratch_types=[
            pltpu.VMEM((BPW,), jnp.int32),
            pltpu.VMEM((BPW,), jnp.int32),
            pltpu.VMEM((BPW, DIM), jnp.float32),
            pltpu.VMEM((BPW, DIM), jnp.float32),
            pltpu.VMEM((BPW,), jnp.float32),
            pltpu.SemaphoreType.DMA,
        ],
        compiler_params=pltpu.CompilerParams(
            needs_layout_passes=False, use_tc_tiling_on_sc=False),
    )
    return f(user_ids.astype(jnp.int32), item_ids.astype(jnp.int32),
             user_table, item_table)


# speedup vs baseline: 5.3683x; 5.3683x over previous
import functools

import jax
import jax.numpy as jnp
from jax import lax
from jax.experimental import pallas as pl
from jax.experimental.pallas import tpu as pltpu
from jax.experimental.pallas import tpu_sc as plsc

BATCH = 16384
DIM = 32
VOCAB = 1_000_000

_info = plsc.get_sparse_core_info()
NC, NS, NL = _info.num_cores, _info.num_subcores, _info.num_lanes
NW = NC * NS
BPW = BATCH // NW
RND = 16
NJT = RND // 8
NROUND = BPW // RND


def _body(uid_hbm, iid_hbm, ut_hbm, it_hbm, out_hbm,
          uid_v, iid_v, ubuf, ibuf, out_v, sem):
    wid = lax.axis_index("s") * NC + lax.axis_index("c")
    base = wid * BPW

    pltpu.sync_copy(uid_hbm.at[pl.ds(base, BPW)], uid_v)
    pltpu.sync_copy(iid_hbm.at[pl.ds(base, BPW)], iid_v)

    iota = lax.iota(jnp.int32, NL)
    kk16 = (iota & 7) * 16
    jt_lo = iota >> 3

    def fire(vec, tbl, buf):
        for k in range(RND):
            r = vec[k]
            off = (r // 16) * 16
            pltpu.async_copy(
                tbl.at[:, :, pl.ds(off, 16)],
                buf.at[:, k // 8, :, pl.ds((k % 8) * 16, 16)],
                sem)

    def drain(tbl, buf):
        for k in range(RND):
            pltpu.make_async_copy(
                tbl.at[:, :, pl.ds(0, 16)],
                buf.at[:, k // 8, :, pl.ds((k % 8) * 16, 16)],
                sem).wait()

    def step(g, carry):
        sl = pl.ds(g * RND, NL)
        uvec = uid_v[sl]
        ivec = iid_v[sl]
        fire(uvec, ut_hbm, ubuf)
        fire(ivec, it_hbm, ibuf)
        drain(ut_hbm, ubuf)
        drain(it_hbm, ibuf)
        lane_u = kk16 + (uvec & 15)
        lane_i = kk16 + (ivec & 15)
        acc = jnp.zeros((NL,), jnp.float32)
        for d in range(DIM):
            iv = jnp.full((NL,), d // 8, jnp.int32)
            sv = jnp.full((NL,), d % 8, jnp.int32)
            u = plsc.load_gather(ubuf, [iv, jt_lo, sv, lane_u])
            v = plsc.load_gather(ibuf, [iv, jt_lo, sv, lane_i])
            acc = acc + u * v
        out_v[sl] = acc
        return carry

    lax.fori_loop(0, NROUND, step, 0)

    pltpu.sync_copy(out_v, out_hbm.at[pl.ds(base, BPW)])


def kernel(user_ids, item_ids, user_table, item_table):
    mesh = plsc.VectorSubcoreMesh(core_axis_name="c", subcore_axis_name="s")
    f = pl.kernel(
        _body,
        mesh=mesh,
        out_type=jax.ShapeDtypeStruct((BATCH,), jnp.float32),
        scratch_types=[
            pltpu.VMEM((BPW,), jnp.int32),
            pltpu.VMEM((BPW,), jnp.int32),
            pltpu.VMEM((4, NJT, 8, 128), jnp.float32),
            pltpu.VMEM((4, NJT, 8, 128), jnp.float32),
            pltpu.VMEM((BPW,), jnp.float32),
            pltpu.SemaphoreType.DMA,
        ],
        compiler_params=pltpu.CompilerParams(
            needs_layout_passes=False, use_tc_tiling_on_sc=True),
    )
    ut3 = user_table.T.reshape(4, 8, VOCAB)
    it3 = item_table.T.reshape(4, 8, VOCAB)
    return f(user_ids.astype(jnp.int32), item_ids.astype(jnp.int32), ut3, it3)
